# baseline (speedup 1.0000x reference)
import functools
import jax
import jax.numpy as jnp
from jax import lax
from jax.experimental import pallas as pl
from jax.experimental.pallas import tpu as pltpu
from jax.experimental.pallas import tpu_sc as plsc

_B, _N, _C, _D = 256, 64, 32, 128
_GT = 8
_B_SC = 224
_TREES_PER_SUBCORE = _B_SC // 32
_L = 16


def _bcast_lane(vec, j):
    dn = lax.GatherDimensionNumbers(offset_dims=(), collapsed_slice_dims=(0,),
                                    start_index_map=(0,))
    idx = jnp.full((_L, 1), j, jnp.int32)
    return lax.gather(vec, idx, dn, slice_sizes=(1,),
                      mode=lax.GatherScatterMode.PROMISE_IN_BOUNDS)


def _sc_body(parent_hbm, idx_hbm, hs_hbm, hw_hbm, tree_v, idx_v, hs_v, hw_v):
    n, c, d, l = _N, _C, _D, _L
    wid = lax.axis_index("s") * 2 + lax.axis_index("c")
    zero = jnp.zeros((l,), jnp.float32)
    lane = lax.iota(jnp.int32, l)
    views = [tree_v.at[pl.ds(k * l, (n - 1) * d + l)] for k in range(d // l)]

    def tree_loop(t, carry):
        b = wid * _TREES_PER_SUBCORE + t
        pltpu.sync_copy(parent_hbm.at[b], tree_v)
        pltpu.sync_copy(idx_hbm.at[b], idx_v)
        for k in range(d // l):
            tree_v[pl.ds(k * l, l)] = zero

        def node_loop(nn, carry2):
            base = nn * c
            halves = []
            for h in range(2):
                iv = idx_v[pl.ds(base + h * l, l)]
                halves.append(iv << 7)
            accs = [jnp.zeros((l,), jnp.float32) for _ in range(d // l)]
            accw = [jnp.zeros((l,), jnp.float32) for _ in range(d // l)]
            for h, siv in reversed(list(enumerate(halves))):
                for j in reversed(range(l)):
                    cpos = h * l + j
                    addr = _bcast_lane(siv, j) + lane
                    for k in range(d // l):
                        g = plsc.load_gather(views[k], [addr])
                        accs[k] = accs[k] + g
                        if cpos != 0:
                            accw[k] = accw[k] + accs[k]
            for k in range(d // l):
                hs_v[pl.ds(nn * d + k * l, l)] = accs[k]
                hw_v[pl.ds(nn * d + k * l, l)] = accw[k]
            return carry2

        lax.fori_loop(0, n, node_loop, 0)
        pltpu.sync_copy(hs_v, hs_hbm.at[b])
        pltpu.sync_copy(hw_v, hw_hbm.at[b])
        return carry

    lax.fori_loop(0, _TREES_PER_SUBCORE, tree_loop, 0)


def _sc_gather(parent2d, idx2d):
    b, nd = parent2d.shape
    mesh = plsc.VectorSubcoreMesh(core_axis_name="c", subcore_axis_name="s")
    f = functools.partial(
        pl.kernel,
        out_type=(
            jax.ShapeDtypeStruct((b, nd), jnp.float32),
            jax.ShapeDtypeStruct((b, nd), jnp.float32),
        ),
        mesh=mesh,
        compiler_params=pltpu.CompilerParams(needs_layout_passes=False),
        scratch_types=[
            pltpu.VMEM((nd,), jnp.float32),
            pltpu.VMEM((_N * _C,), jnp.int32),
            pltpu.VMEM((nd,), jnp.float32),
            pltpu.VMEM((nd,), jnp.float32),
        ],
    )(_sc_body)
    return f(parent2d, idx2d)


def _tc_full_body(parent_ref, idx_ref, wt_ref, wl_ref, wr_ref, bias_ref,
                  gamma_ref, beta_ref, out_ref):
    gt, n, d, c = _GT, _N, _D, _C
    rows = gt * n

    idx = idx_ref[...]
    mask = (idx != 0).astype(jnp.float32)
    ns = jnp.sum(mask, axis=1, keepdims=True)
    c_iota_i = jax.lax.broadcasted_iota(jnp.int32, (rows, c), 1)
    c_iota = c_iota_i.astype(jnp.float32)
    safe = jnp.where(ns == 1.0, 1.0, ns - 1.0)
    er_gen = c_iota * mask / safe
    er_single = jnp.where(c_iota_i == 0, 0.5, 0.0)
    er = jnp.where(ns == 1.0, er_single, er_gen)
    cl = mask * (1.0 - er)
    cr = mask * er

    m_iota = jax.lax.broadcasted_iota(jnp.int32, (rows, n), 1)
    s_l = jnp.zeros((rows, n), jnp.float32)
    s_r = jnp.zeros((rows, n), jnp.float32)
    for j in range(c):
        idx_j = idx[:, j:j + 1]
        eq = (m_iota == idx_j).astype(jnp.float32)
        s_l = s_l + cl[:, j:j + 1] * eq
        s_r = s_r + cr[:, j:j + 1] * eq

    parent = parent_ref[...]
    hl_parts = []
    hr_parts = []
    for g in range(gt):
        p_g = parent[g]
        sl_g = s_l[g * n:(g + 1) * n, :]
        sr_g = s_r[g * n:(g + 1) * n, :]
        hl_parts.append(jnp.dot(sl_g, p_g, preferred_element_type=jnp.float32))
        hr_parts.append(jnp.dot(sr_g, p_g, preferred_element_type=jnp.float32))
    h_l = jnp.concatenate(hl_parts, axis=0)
    h_r = jnp.concatenate(hr_parts, axis=0)

    p_flat = parent.reshape(rows, d)
    x = (jnp.dot(p_flat, wt_ref[...], preferred_element_type=jnp.float32)
         + jnp.dot(h_l, wl_ref[...], preferred_element_type=jnp.float32)
         + jnp.dot(h_r, wr_ref[...], preferred_element_type=jnp.float32)
         + bias_ref[...] + p_flat)

    mu = jnp.mean(x, axis=1, keepdims=True)
    var = jnp.mean(x * x, axis=1, keepdims=True) - mu * mu
    y = (x - mu) * jax.lax.rsqrt(var + 1e-5) * gamma_ref[...] + beta_ref[...]
    y = jnp.maximum(y, 0.0)
    out_ref[...] = jnp.max(y.reshape(gt, n, d), axis=1)


def _tc_full(parent3d, idx_flat, w_t, w_l, w_r, bias2, gamma2, beta2):
    b, n, d, c, gt = parent3d.shape[0], _N, _D, _C, _GT
    grid = (b // gt,)
    return pl.pallas_call(
        _tc_full_body,
        grid=grid,
        in_specs=[
            pl.BlockSpec((gt, n, d), lambda i: (i, 0, 0)),
            pl.BlockSpec((gt * n, c), lambda i: (i, 0)),
            pl.BlockSpec((d, d), lambda i: (0, 0)),
            pl.BlockSpec((d, d), lambda i: (0, 0)),
            pl.BlockSpec((d, d), lambda i: (0, 0)),
            pl.BlockSpec((1, d), lambda i: (0, 0)),
            pl.BlockSpec((1, d), lambda i: (0, 0)),
            pl.BlockSpec((1, d), lambda i: (0, 0)),
        ],
        out_specs=pl.BlockSpec((gt, d), lambda i: (i, 0)),
        out_shape=jax.ShapeDtypeStruct((b, d), jnp.float32),
    )(parent3d, idx_flat, w_t, w_l, w_r, bias2, gamma2, beta2)


def _dense_body(p_ref, hs_ref, hw_ref, idx_ref, wt_ref, wl_ref, wr_ref,
                bias_ref, gamma_ref, beta_ref, out_ref):
    gt, n, d = _GT, _N, _D
    p = p_ref[...]
    hs = hs_ref[...]
    hw = hw_ref[...]

    idx = idx_ref[...]
    mask = (idx != 0).astype(jnp.float32)
    ns = jnp.sum(mask, axis=1, keepdims=True)
    ns1 = ns == 1.0
    m0 = mask[:, 0:1]
    rinv = 1.0 / jnp.where(ns1, 2.0, ns - 1.0)
    ca = jnp.where(ns1, 0.5 * m0, 0.0)
    cb = jnp.where(ns1, 0.0, rinv)
    h_r = ca * hs + cb * hw

    w_rl = wr_ref[...] - wl_ref[...]
    x = (jnp.dot(p, wt_ref[...], preferred_element_type=jnp.float32)
         + jnp.dot(hs, wl_ref[...], preferred_element_type=jnp.float32)
         + jnp.dot(h_r, w_rl, preferred_element_type=jnp.float32)
         + bias_ref[...] + p)

    mu = jnp.mean(x, axis=1, keepdims=True)
    var = jnp.mean(x * x, axis=1, keepdims=True) - mu * mu
    y = (x - mu) * jax.lax.rsqrt(var + 1e-5) * gamma_ref[...] + beta_ref[...]
    y = jnp.maximum(y, 0.0)
    out_ref[...] = jnp.max(y.reshape(gt, n, d), axis=1)


def _dense(p_flat, hs_flat, hw_flat, idx_flat, w_t, w_l, w_r, bias2, gamma2,
           beta2):
    n, d, c, gt = _N, _D, _C, _GT
    b = p_flat.shape[0] // n
    rows = gt * n
    grid = (b // gt,)
    row_spec = pl.BlockSpec((rows, d), lambda i: (i, 0))
    idx_spec = pl.BlockSpec((rows, c), lambda i: (i, 0))
    w_spec = pl.BlockSpec((d, d), lambda i: (0, 0))
    v_spec = pl.BlockSpec((1, d), lambda i: (0, 0))
    return pl.pallas_call(
        _dense_body,
        grid=grid,
        in_specs=[row_spec, row_spec, row_spec, idx_spec, w_spec, w_spec,
                  w_spec, v_spec, v_spec, v_spec],
        out_specs=pl.BlockSpec((gt, d), lambda i: (i, 0)),
        out_shape=jax.ShapeDtypeStruct((b, d), jnp.float32),
    )(p_flat, hs_flat, hw_flat, idx_flat, w_t, w_l, w_r, bias2, gamma2, beta2)


def kernel(parent_node_embedding, children_index, batch_tree_mask, w_t, w_l,
           w_r, bias, ln_gamma, ln_beta):
    del batch_tree_mask
    b, n, d, c, bsc = _B, _N, _D, _C, _B_SC
    bias2 = bias.reshape(1, d)
    gamma2 = ln_gamma.reshape(1, d)
    beta2 = ln_beta.reshape(1, d)

    p_lo = parent_node_embedding[:bsc]
    i_lo = children_index[:bsc]
    p_hi = parent_node_embedding[bsc:]
    i_hi = children_index[bsc:]

    hs2d, hw2d = _sc_gather(p_lo.reshape(bsc, n * d),
                            i_lo.reshape(bsc, n * c))

    out_hi = _tc_full(p_hi, i_hi.reshape((b - bsc) * n, c),
                      w_t, w_l, w_r, bias2, gamma2, beta2)

    out_lo = _dense(p_lo.reshape(bsc * n, d),
                    hs2d.reshape(bsc * n, d),
                    hw2d.reshape(bsc * n, d),
                    i_lo.reshape(bsc * n, c),
                    w_t, w_l, w_r, bias2, gamma2, beta2)

    return jnp.concatenate([out_lo, out_hi], axis=0)


# speedup vs baseline: 1.1411x; 1.1411x over previous
import functools
import jax
import jax.numpy as jnp
from jax import lax
from jax.experimental import pallas as pl
from jax.experimental.pallas import tpu as pltpu
from jax.experimental.pallas import tpu_sc as plsc

_B, _N, _C, _D = 256, 64, 32, 128
_GT = 8
_B_SC = 192
_TREES_PER_SUBCORE = _B_SC // 32
_L = 16


def _bcast_lane(vec, j):
    dn = lax.GatherDimensionNumbers(offset_dims=(), collapsed_slice_dims=(0,),
                                    start_index_map=(0,))
    idx = jnp.full((_L, 1), j, jnp.int32)
    return lax.gather(vec, idx, dn, slice_sizes=(1,),
                      mode=lax.GatherScatterMode.PROMISE_IN_BOUNDS)


def _sc_body(parent_hbm, idx_hbm, hs_hbm, hw_hbm, tree_v, idx_v, hs_v, hw_v):
    n, c, d, l = _N, _C, _D, _L
    wid = lax.axis_index("s") * 2 + lax.axis_index("c")
    zero = jnp.zeros((l,), jnp.float32)
    lane = lax.iota(jnp.int32, l)
    views = [tree_v.at[pl.ds(k * l, (n - 1) * d + l)] for k in range(d // l)]

    def tree_loop(t, carry):
        b = wid * _TREES_PER_SUBCORE + t
        pltpu.sync_copy(parent_hbm.at[b], tree_v)
        pltpu.sync_copy(idx_hbm.at[b], idx_v)
        for k in range(d // l):
            tree_v[pl.ds(k * l, l)] = zero

        def node_loop(nn, carry2):
            base = nn * c
            halves = []
            for h in range(2):
                iv = idx_v[pl.ds(base + h * l, l)]
                halves.append(iv << 7)
            accs = [jnp.zeros((l,), jnp.float32) for _ in range(d // l)]
            accw = [jnp.zeros((l,), jnp.float32) for _ in range(d // l)]
            for h, siv in reversed(list(enumerate(halves))):
                for j in reversed(range(l)):
                    cpos = h * l + j
                    addr = _bcast_lane(siv, j) + lane
                    for k in range(d // l):
                        g = plsc.load_gather(views[k], [addr])
                        accs[k] = accs[k] + g
                        if cpos != 0:
                            accw[k] = accw[k] + accs[k]
            for k in range(d // l):
                hs_v[pl.ds(nn * d + k * l, l)] = accs[k]
                hw_v[pl.ds(nn * d + k * l, l)] = accw[k]
            return carry2

        lax.fori_loop(0, n, node_loop, 0)
        pltpu.sync_copy(hs_v, hs_hbm.at[b])
        pltpu.sync_copy(hw_v, hw_hbm.at[b])
        return carry

    lax.fori_loop(0, _TREES_PER_SUBCORE, tree_loop, 0)


def _sc_gather(parent2d, idx2d):
    b, nd = parent2d.shape
    mesh = plsc.VectorSubcoreMesh(core_axis_name="c", subcore_axis_name="s")
    f = functools.partial(
        pl.kernel,
        out_type=(
            jax.ShapeDtypeStruct((b, nd), jnp.float32),
            jax.ShapeDtypeStruct((b, nd), jnp.float32),
        ),
        mesh=mesh,
        compiler_params=pltpu.CompilerParams(needs_layout_passes=False),
        scratch_types=[
            pltpu.VMEM((nd,), jnp.float32),
            pltpu.VMEM((_N * _C,), jnp.int32),
            pltpu.VMEM((nd,), jnp.float32),
            pltpu.VMEM((nd,), jnp.float32),
        ],
    )(_sc_body)
    return f(parent2d, idx2d)


def _tc_full_body(parent_ref, idx_ref, wt_ref, wl_ref, wr_ref, bias_ref,
                  gamma_ref, beta_ref, out_ref):
    gt, n, d, c = _GT, _N, _D, _C
    rows = gt * n

    idx = idx_ref[...]
    mask = (idx != 0).astype(jnp.float32)
    ns = jnp.sum(mask, axis=1, keepdims=True)
    c_iota_i = jax.lax.broadcasted_iota(jnp.int32, (rows, c), 1)
    c_iota = c_iota_i.astype(jnp.float32)
    safe = jnp.where(ns == 1.0, 1.0, ns - 1.0)
    er_gen = c_iota * mask / safe
    er_single = jnp.where(c_iota_i == 0, 0.5, 0.0)
    er = jnp.where(ns == 1.0, er_single, er_gen)
    cl = mask * (1.0 - er)
    cr = mask * er

    m_iota = jax.lax.broadcasted_iota(jnp.int32, (rows, n), 1)
    s_l = jnp.zeros((rows, n), jnp.float32)
    s_r = jnp.zeros((rows, n), jnp.float32)
    for j in range(c):
        idx_j = idx[:, j:j + 1]
        eq = (m_iota == idx_j).astype(jnp.float32)
        s_l = s_l + cl[:, j:j + 1] * eq
        s_r = s_r + cr[:, j:j + 1] * eq

    parent = parent_ref[...]
    hl_parts = []
    hr_parts = []
    for g in range(gt):
        p_g = parent[g]
        sl_g = s_l[g * n:(g + 1) * n, :]
        sr_g = s_r[g * n:(g + 1) * n, :]
        hl_parts.append(jnp.dot(sl_g, p_g, preferred_element_type=jnp.float32))
        hr_parts.append(jnp.dot(sr_g, p_g, preferred_element_type=jnp.float32))
    h_l = jnp.concatenate(hl_parts, axis=0)
    h_r = jnp.concatenate(hr_parts, axis=0)

    p_flat = parent.reshape(rows, d)
    x = (jnp.dot(p_flat, wt_ref[...], preferred_element_type=jnp.float32)
         + jnp.dot(h_l, wl_ref[...], preferred_element_type=jnp.float32)
         + jnp.dot(h_r, wr_ref[...], preferred_element_type=jnp.float32)
         + bias_ref[...] + p_flat)

    mu = jnp.mean(x, axis=1, keepdims=True)
    var = jnp.mean(x * x, axis=1, keepdims=True) - mu * mu
    y = (x - mu) * jax.lax.rsqrt(var + 1e-5) * gamma_ref[...] + beta_ref[...]
    y = jnp.maximum(y, 0.0)
    out_ref[...] = jnp.max(y.reshape(gt, n, d), axis=1)


def _tc_full(parent3d, idx_flat, w_t, w_l, w_r, bias2, gamma2, beta2):
    b, n, d, c, gt = parent3d.shape[0], _N, _D, _C, _GT
    grid = (b // gt,)
    return pl.pallas_call(
        _tc_full_body,
        grid=grid,
        in_specs=[
            pl.BlockSpec((gt, n, d), lambda i: (i, 0, 0)),
            pl.BlockSpec((gt * n, c), lambda i: (i, 0)),
            pl.BlockSpec((d, d), lambda i: (0, 0)),
            pl.BlockSpec((d, d), lambda i: (0, 0)),
            pl.BlockSpec((d, d), lambda i: (0, 0)),
            pl.BlockSpec((1, d), lambda i: (0, 0)),
            pl.BlockSpec((1, d), lambda i: (0, 0)),
            pl.BlockSpec((1, d), lambda i: (0, 0)),
        ],
        out_specs=pl.BlockSpec((gt, d), lambda i: (i, 0)),
        out_shape=jax.ShapeDtypeStruct((b, d), jnp.float32),
    )(parent3d, idx_flat, w_t, w_l, w_r, bias2, gamma2, beta2)


def _dense_body(p_ref, hs_ref, hw_ref, idx_ref, wt_ref, wl_ref, wr_ref,
                bias_ref, gamma_ref, beta_ref, out_ref):
    gt, n, d = _GT, _N, _D
    p = p_ref[...]
    hs = hs_ref[...]
    hw = hw_ref[...]

    idx = idx_ref[...]
    mask = (idx != 0).astype(jnp.float32)
    ns = jnp.sum(mask, axis=1, keepdims=True)
    ns1 = ns == 1.0
    m0 = mask[:, 0:1]
    rinv = 1.0 / jnp.where(ns1, 2.0, ns - 1.0)
    ca = jnp.where(ns1, 0.5 * m0, 0.0)
    cb = jnp.where(ns1, 0.0, rinv)
    h_r = ca * hs + cb * hw

    w_rl = wr_ref[...] - wl_ref[...]
    x = (jnp.dot(p, wt_ref[...], preferred_element_type=jnp.float32)
         + jnp.dot(hs, wl_ref[...], preferred_element_type=jnp.float32)
         + jnp.dot(h_r, w_rl, preferred_element_type=jnp.float32)
         + bias_ref[...] + p)

    mu = jnp.mean(x, axis=1, keepdims=True)
    var = jnp.mean(x * x, axis=1, keepdims=True) - mu * mu
    y = (x - mu) * jax.lax.rsqrt(var + 1e-5) * gamma_ref[...] + beta_ref[...]
    y = jnp.maximum(y, 0.0)
    out_ref[...] = jnp.max(y.reshape(gt, n, d), axis=1)


def _dense(p_flat, hs_flat, hw_flat, idx_flat, w_t, w_l, w_r, bias2, gamma2,
           beta2):
    n, d, c, gt = _N, _D, _C, _GT
    b = p_flat.shape[0] // n
    rows = gt * n
    grid = (b // gt,)
    row_spec = pl.BlockSpec((rows, d), lambda i: (i, 0))
    idx_spec = pl.BlockSpec((rows, c), lambda i: (i, 0))
    w_spec = pl.BlockSpec((d, d), lambda i: (0, 0))
    v_spec = pl.BlockSpec((1, d), lambda i: (0, 0))
    return pl.pallas_call(
        _dense_body,
        grid=grid,
        in_specs=[row_spec, row_spec, row_spec, idx_spec, w_spec, w_spec,
                  w_spec, v_spec, v_spec, v_spec],
        out_specs=pl.BlockSpec((gt, d), lambda i: (i, 0)),
        out_shape=jax.ShapeDtypeStruct((b, d), jnp.float32),
    )(p_flat, hs_flat, hw_flat, idx_flat, w_t, w_l, w_r, bias2, gamma2, beta2)


def kernel(parent_node_embedding, children_index, batch_tree_mask, w_t, w_l,
           w_r, bias, ln_gamma, ln_beta):
    del batch_tree_mask
    b, n, d, c, bsc = _B, _N, _D, _C, _B_SC
    bias2 = bias.reshape(1, d)
    gamma2 = ln_gamma.reshape(1, d)
    beta2 = ln_beta.reshape(1, d)

    p_lo = parent_node_embedding[:bsc]
    i_lo = children_index[:bsc]
    p_hi = parent_node_embedding[bsc:]
    i_hi = children_index[bsc:]

    hs2d, hw2d = _sc_gather(p_lo.reshape(bsc, n * d),
                            i_lo.reshape(bsc, n * c))

    out_hi = _tc_full(p_hi, i_hi.reshape((b - bsc) * n, c),
                      w_t, w_l, w_r, bias2, gamma2, beta2)

    out_lo = _dense(p_lo.reshape(bsc * n, d),
                    hs2d.reshape(bsc * n, d),
                    hw2d.reshape(bsc * n, d),
                    i_lo.reshape(bsc * n, c),
                    w_t, w_l, w_r, bias2, gamma2, beta2)

    return jnp.concatenate([out_lo, out_hi], axis=0)


# speedup vs baseline: 1.1422x; 1.0009x over previous
import functools
import jax
import jax.numpy as jnp
from jax import lax
from jax.experimental import pallas as pl
from jax.experimental.pallas import tpu as pltpu
from jax.experimental.pallas import tpu_sc as plsc

_B, _N, _C, _D = 256, 64, 32, 128
_GT = 8
_B_SC = 192
_TREES_PER_SUBCORE = _B_SC // 32
_L = 16


def _bcast_lane(vec, j):
    dn = lax.GatherDimensionNumbers(offset_dims=(), collapsed_slice_dims=(0,),
                                    start_index_map=(0,))
    idx = jnp.full((_L, 1), j, jnp.int32)
    return lax.gather(vec, idx, dn, slice_sizes=(1,),
                      mode=lax.GatherScatterMode.PROMISE_IN_BOUNDS)


def _sc_body(parent_hbm, idx_hbm, hs_hbm, hw_hbm, tree_v, idx_v, hs_v, hw_v):
    n, c, d, l = _N, _C, _D, _L
    wid = lax.axis_index("s") * 2 + lax.axis_index("c")
    zero = jnp.zeros((l,), jnp.float32)
    lane = lax.iota(jnp.int32, l)
    views = [tree_v.at[pl.ds(k * l, (n - 1) * d + l)] for k in range(d // l)]

    def tree_loop(t, carry):
        b = wid * _TREES_PER_SUBCORE + t
        pltpu.sync_copy(parent_hbm.at[b], tree_v)
        pltpu.sync_copy(idx_hbm.at[b], idx_v)
        for k in range(d // l):
            tree_v[pl.ds(k * l, l)] = zero

        def node_loop(nn, carry2):
            base = nn * c
            halves = []
            for h in range(2):
                iv = idx_v[pl.ds(base + h * l, l)]
                halves.append(iv << 7)
            accs = [jnp.zeros((l,), jnp.float32) for _ in range(d // l)]
            accw = [jnp.zeros((l,), jnp.float32) for _ in range(d // l)]
            for h, siv in reversed(list(enumerate(halves))):
                for j in reversed(range(l)):
                    cpos = h * l + j
                    addr = _bcast_lane(siv, j) + lane
                    for k in range(d // l):
                        g = plsc.load_gather(views[k], [addr])
                        accs[k] = accs[k] + g
                        if cpos != 0:
                            accw[k] = accw[k] + accs[k]
            for k in range(d // l):
                hs_v[pl.ds(nn * d + k * l, l)] = accs[k]
                hw_v[pl.ds(nn * d + k * l, l)] = accw[k]
            return carry2

        lax.fori_loop(0, n, node_loop, 0)
        pltpu.sync_copy(hs_v, hs_hbm.at[b])
        pltpu.sync_copy(hw_v, hw_hbm.at[b])
        return carry

    lax.fori_loop(0, _TREES_PER_SUBCORE, tree_loop, 0)


def _sc_gather(parent2d, idx2d):
    b, nd = parent2d.shape
    mesh = plsc.VectorSubcoreMesh(core_axis_name="c", subcore_axis_name="s")
    f = functools.partial(
        pl.kernel,
        out_type=(
            jax.ShapeDtypeStruct((b, nd), jnp.float32),
            jax.ShapeDtypeStruct((b, nd), jnp.float32),
        ),
        mesh=mesh,
        compiler_params=pltpu.CompilerParams(needs_layout_passes=False),
        scratch_types=[
            pltpu.VMEM((nd,), jnp.float32),
            pltpu.VMEM((_N * _C,), jnp.int32),
            pltpu.VMEM((nd,), jnp.float32),
            pltpu.VMEM((nd,), jnp.float32),
        ],
    )(_sc_body)
    return f(parent2d, idx2d)


def _tc_full_body(parent_ref, idxt_ref, wt_ref, wl_ref, wr_ref,
                  bias_ref, gamma_ref, beta_ref, out_ref):
    gt, n, d, c = _GT, _N, _D, _C
    rows = gt * n

    idxt = idxt_ref[...]
    mask = (idxt != 0).astype(jnp.float32)
    ns = jnp.sum(mask, axis=0, keepdims=True)
    c_iota_i = jax.lax.broadcasted_iota(jnp.int32, (c, rows), 0)
    c_iota = c_iota_i.astype(jnp.float32)
    safe = jnp.where(ns == 1.0, 1.0, ns - 1.0)
    er_gen = c_iota * mask / safe
    er_single = jnp.where(c_iota_i == 0, 0.5, 0.0)
    er = jnp.where(ns == 1.0, er_single, er_gen)
    cl = mask * (1.0 - er)
    cr = mask * er

    m_iota = jax.lax.broadcasted_iota(jnp.int32, (n, rows), 0)
    st_l = jnp.zeros((n, rows), jnp.float32)
    st_r = jnp.zeros((n, rows), jnp.float32)
    for j in range(c):
        eq = (m_iota == idxt[j:j + 1, :]).astype(jnp.float32)
        st_l = st_l + cl[j:j + 1, :] * eq
        st_r = st_r + cr[j:j + 1, :] * eq

    parent = parent_ref[...]
    dn_t = (((0,), (0,)), ((), ()))
    hl_parts = []
    hr_parts = []
    for g in range(gt):
        p_g = parent[g]
        sl_g = st_l[:, g * n:(g + 1) * n]
        sr_g = st_r[:, g * n:(g + 1) * n]
        hl_parts.append(lax.dot_general(sl_g, p_g, dn_t,
                                        preferred_element_type=jnp.float32))
        hr_parts.append(lax.dot_general(sr_g, p_g, dn_t,
                                        preferred_element_type=jnp.float32))
    h_l = jnp.concatenate(hl_parts, axis=0)
    h_r = jnp.concatenate(hr_parts, axis=0)

    p_flat = parent.reshape(rows, d)
    x = (jnp.dot(p_flat, wt_ref[...], preferred_element_type=jnp.float32)
         + jnp.dot(h_l, wl_ref[...], preferred_element_type=jnp.float32)
         + jnp.dot(h_r, wr_ref[...], preferred_element_type=jnp.float32)
         + bias_ref[...] + p_flat)

    mu = jnp.mean(x, axis=1, keepdims=True)
    var = jnp.mean(x * x, axis=1, keepdims=True) - mu * mu
    y = (x - mu) * jax.lax.rsqrt(var + 1e-5) * gamma_ref[...] + beta_ref[...]
    y = jnp.maximum(y, 0.0)
    out_ref[...] = jnp.max(y.reshape(gt, n, d), axis=1)


def _tc_full(parent3d, idx_t, w_t, w_l, w_r, bias2, gamma2, beta2):
    b, n, d, c, gt = parent3d.shape[0], _N, _D, _C, _GT
    grid = (b // gt,)
    return pl.pallas_call(
        _tc_full_body,
        grid=grid,
        in_specs=[
            pl.BlockSpec((gt, n, d), lambda i: (i, 0, 0)),
            pl.BlockSpec((c, gt * n), lambda i: (0, i)),
            pl.BlockSpec((d, d), lambda i: (0, 0)),
            pl.BlockSpec((d, d), lambda i: (0, 0)),
            pl.BlockSpec((d, d), lambda i: (0, 0)),
            pl.BlockSpec((1, d), lambda i: (0, 0)),
            pl.BlockSpec((1, d), lambda i: (0, 0)),
            pl.BlockSpec((1, d), lambda i: (0, 0)),
        ],
        out_specs=pl.BlockSpec((gt, d), lambda i: (i, 0)),
        out_shape=jax.ShapeDtypeStruct((b, d), jnp.float32),
    )(parent3d, idx_t, w_t, w_l, w_r, bias2, gamma2, beta2)


def _dense_body(p_ref, hs_ref, hw_ref, idx_ref, wt_ref, wl_ref, wr_ref,
                bias_ref, gamma_ref, beta_ref, out_ref):
    gt, n, d = _GT, _N, _D
    p = p_ref[...]
    hs = hs_ref[...]
    hw = hw_ref[...]

    idx = idx_ref[...]
    mask = (idx != 0).astype(jnp.float32)
    ns = jnp.sum(mask, axis=1, keepdims=True)
    ns1 = ns == 1.0
    m0 = mask[:, 0:1]
    rinv = 1.0 / jnp.where(ns1, 2.0, ns - 1.0)
    ca = jnp.where(ns1, 0.5 * m0, 0.0)
    cb = jnp.where(ns1, 0.0, rinv)
    h_r = ca * hs + cb * hw

    w_rl = wr_ref[...] - wl_ref[...]
    x = (jnp.dot(p, wt_ref[...], preferred_element_type=jnp.float32)
         + jnp.dot(hs, wl_ref[...], preferred_element_type=jnp.float32)
         + jnp.dot(h_r, w_rl, preferred_element_type=jnp.float32)
         + bias_ref[...] + p)

    mu = jnp.mean(x, axis=1, keepdims=True)
    var = jnp.mean(x * x, axis=1, keepdims=True) - mu * mu
    y = (x - mu) * jax.lax.rsqrt(var + 1e-5) * gamma_ref[...] + beta_ref[...]
    y = jnp.maximum(y, 0.0)
    out_ref[...] = jnp.max(y.reshape(gt, n, d), axis=1)


def _dense(p_flat, hs_flat, hw_flat, idx_flat, w_t, w_l, w_r, bias2, gamma2,
           beta2):
    n, d, c, gt = _N, _D, _C, _GT
    b = p_flat.shape[0] // n
    rows = gt * n
    grid = (b // gt,)
    row_spec = pl.BlockSpec((rows, d), lambda i: (i, 0))
    idx_spec = pl.BlockSpec((rows, c), lambda i: (i, 0))
    w_spec = pl.BlockSpec((d, d), lambda i: (0, 0))
    v_spec = pl.BlockSpec((1, d), lambda i: (0, 0))
    return pl.pallas_call(
        _dense_body,
        grid=grid,
        in_specs=[row_spec, row_spec, row_spec, idx_spec, w_spec, w_spec,
                  w_spec, v_spec, v_spec, v_spec],
        out_specs=pl.BlockSpec((gt, d), lambda i: (i, 0)),
        out_shape=jax.ShapeDtypeStruct((b, d), jnp.float32),
    )(p_flat, hs_flat, hw_flat, idx_flat, w_t, w_l, w_r, bias2, gamma2, beta2)


def kernel(parent_node_embedding, children_index, batch_tree_mask, w_t, w_l,
           w_r, bias, ln_gamma, ln_beta):
    del batch_tree_mask
    b, n, d, c, bsc = _B, _N, _D, _C, _B_SC
    bias2 = bias.reshape(1, d)
    gamma2 = ln_gamma.reshape(1, d)
    beta2 = ln_beta.reshape(1, d)

    p_lo = parent_node_embedding[:bsc]
    i_lo = children_index[:bsc]
    p_hi = parent_node_embedding[bsc:]
    i_hi = children_index[bsc:]

    hs2d, hw2d = _sc_gather(p_lo.reshape(bsc, n * d),
                            i_lo.reshape(bsc, n * c))

    out_hi = _tc_full(p_hi, i_hi.reshape((b - bsc) * n, c).T,
                      w_t, w_l, w_r, bias2, gamma2, beta2)

    out_lo = _dense(p_lo.reshape(bsc * n, d),
                    hs2d.reshape(bsc * n, d),
                    hw2d.reshape(bsc * n, d),
                    i_lo.reshape(bsc * n, c),
                    w_t, w_l, w_r, bias2, gamma2, beta2)

    return jnp.concatenate([out_lo, out_hi], axis=0)


# speedup vs baseline: 1.3808x; 1.2089x over previous
import functools
import jax
import jax.numpy as jnp
from jax import lax
from jax.experimental import pallas as pl
from jax.experimental.pallas import tpu as pltpu
from jax.experimental.pallas import tpu_sc as plsc

_B, _N, _C, _D = 256, 64, 32, 128
_GT = 8
_B_SC = 160
_TREES_PER_SUBCORE = _B_SC // 32
_L = 16


def _bcast_lane(vec, j):
    dn = lax.GatherDimensionNumbers(offset_dims=(), collapsed_slice_dims=(0,),
                                    start_index_map=(0,))
    idx = jnp.full((_L, 1), j, jnp.int32)
    return lax.gather(vec, idx, dn, slice_sizes=(1,),
                      mode=lax.GatherScatterMode.PROMISE_IN_BOUNDS)


def _sc_body(parent_hbm, idx_hbm, hs_hbm, hw_hbm, tree_v, idx_v, hs_v, hw_v):
    n, c, d, l = _N, _C, _D, _L
    wid = lax.axis_index("s") * 2 + lax.axis_index("c")
    zero = jnp.zeros((l,), jnp.float32)
    lane = lax.iota(jnp.int32, l)
    views = [tree_v.at[pl.ds(k * l, (n - 1) * d + l)] for k in range(d // l)]

    def tree_loop(t, carry):
        b = wid * _TREES_PER_SUBCORE + t
        pltpu.sync_copy(parent_hbm.at[b], tree_v)
        pltpu.sync_copy(idx_hbm.at[b], idx_v)
        for k in range(d // l):
            tree_v[pl.ds(k * l, l)] = zero

        def node_loop(nn, carry2):
            base = nn * c
            halves = []
            for h in range(2):
                iv = idx_v[pl.ds(base + h * l, l)]
                halves.append(iv << 7)
            accs = [jnp.zeros((l,), jnp.float32) for _ in range(d // l)]
            accw = [jnp.zeros((l,), jnp.float32) for _ in range(d // l)]
            for h, siv in reversed(list(enumerate(halves))):
                for j in reversed(range(l)):
                    cpos = h * l + j
                    addr = _bcast_lane(siv, j) + lane
                    for k in range(d // l):
                        g = plsc.load_gather(views[k], [addr])
                        accs[k] = accs[k] + g
                        if cpos != 0:
                            accw[k] = accw[k] + accs[k]
            for k in range(d // l):
                hs_v[pl.ds(nn * d + k * l, l)] = accs[k]
                hw_v[pl.ds(nn * d + k * l, l)] = accw[k]
            return carry2

        lax.fori_loop(0, n, node_loop, 0)
        pltpu.sync_copy(hs_v, hs_hbm.at[b])
        pltpu.sync_copy(hw_v, hw_hbm.at[b])
        return carry

    lax.fori_loop(0, _TREES_PER_SUBCORE, tree_loop, 0)


def _sc_gather(parent2d, idx2d):
    b, nd = parent2d.shape
    mesh = plsc.VectorSubcoreMesh(core_axis_name="c", subcore_axis_name="s")
    f = functools.partial(
        pl.kernel,
        out_type=(
            jax.ShapeDtypeStruct((b, nd), jnp.float32),
            jax.ShapeDtypeStruct((b, nd), jnp.float32),
        ),
        mesh=mesh,
        compiler_params=pltpu.CompilerParams(needs_layout_passes=False),
        scratch_types=[
            pltpu.VMEM((nd,), jnp.float32),
            pltpu.VMEM((_N * _C,), jnp.int32),
            pltpu.VMEM((nd,), jnp.float32),
            pltpu.VMEM((nd,), jnp.float32),
        ],
    )(_sc_body)
    return f(parent2d, idx2d)


def _tc_full_body(parent_ref, idxt_ref, wt_ref, wl_ref, wr_ref,
                  bias_ref, gamma_ref, beta_ref, out_ref):
    gt, n, d, c = _GT, _N, _D, _C
    rows = gt * n

    idxt = idxt_ref[...]
    mask = (idxt != 0).astype(jnp.float32)
    ns = jnp.sum(mask, axis=0, keepdims=True)
    c_iota_i = jax.lax.broadcasted_iota(jnp.int32, (c, rows), 0)
    c_iota = c_iota_i.astype(jnp.float32)
    safe = jnp.where(ns == 1.0, 1.0, ns - 1.0)
    er_gen = c_iota * mask / safe
    er_single = jnp.where(c_iota_i == 0, 0.5, 0.0)
    er = jnp.where(ns == 1.0, er_single, er_gen)
    cl = mask * (1.0 - er)
    cr = mask * er

    m_iota = jax.lax.broadcasted_iota(jnp.int32, (n, rows), 0)
    st_l = jnp.zeros((n, rows), jnp.float32)
    st_r = jnp.zeros((n, rows), jnp.float32)
    for j in range(c):
        eq = (m_iota == idxt[j:j + 1, :]).astype(jnp.float32)
        st_l = st_l + cl[j:j + 1, :] * eq
        st_r = st_r + cr[j:j + 1, :] * eq

    parent = parent_ref[...]
    dn_t = (((0,), (0,)), ((), ()))
    hl_parts = []
    hr_parts = []
    for g in range(gt):
        p_g = parent[g]
        sl_g = st_l[:, g * n:(g + 1) * n]
        sr_g = st_r[:, g * n:(g + 1) * n]
        hl_parts.append(lax.dot_general(sl_g, p_g, dn_t,
                                        preferred_element_type=jnp.float32))
        hr_parts.append(lax.dot_general(sr_g, p_g, dn_t,
                                        preferred_element_type=jnp.float32))
    h_l = jnp.concatenate(hl_parts, axis=0)
    h_r = jnp.concatenate(hr_parts, axis=0)

    p_flat = parent.reshape(rows, d)
    x = (jnp.dot(p_flat, wt_ref[...], preferred_element_type=jnp.float32)
         + jnp.dot(h_l, wl_ref[...], preferred_element_type=jnp.float32)
         + jnp.dot(h_r, wr_ref[...], preferred_element_type=jnp.float32)
         + bias_ref[...] + p_flat)

    mu = jnp.mean(x, axis=1, keepdims=True)
    var = jnp.mean(x * x, axis=1, keepdims=True) - mu * mu
    y = (x - mu) * jax.lax.rsqrt(var + 1e-5) * gamma_ref[...] + beta_ref[...]
    y = jnp.maximum(y, 0.0)
    out_ref[...] = jnp.max(y.reshape(gt, n, d), axis=1)


def _tc_full(parent3d, idx_t, w_t, w_l, w_r, bias2, gamma2, beta2):
    b, n, d, c, gt = parent3d.shape[0], _N, _D, _C, _GT
    grid = (b // gt,)
    return pl.pallas_call(
        _tc_full_body,
        grid=grid,
        in_specs=[
            pl.BlockSpec((gt, n, d), lambda i: (i, 0, 0)),
            pl.BlockSpec((c, gt * n), lambda i: (0, i)),
            pl.BlockSpec((d, d), lambda i: (0, 0)),
            pl.BlockSpec((d, d), lambda i: (0, 0)),
            pl.BlockSpec((d, d), lambda i: (0, 0)),
            pl.BlockSpec((1, d), lambda i: (0, 0)),
            pl.BlockSpec((1, d), lambda i: (0, 0)),
            pl.BlockSpec((1, d), lambda i: (0, 0)),
        ],
        out_specs=pl.BlockSpec((gt, d), lambda i: (i, 0)),
        out_shape=jax.ShapeDtypeStruct((b, d), jnp.float32),
    )(parent3d, idx_t, w_t, w_l, w_r, bias2, gamma2, beta2)


def _dense_body(p_ref, hs_ref, hw_ref, idx_ref, wt_ref, wl_ref, wr_ref,
                bias_ref, gamma_ref, beta_ref, out_ref):
    gt, n, d = _GT, _N, _D
    p = p_ref[...]
    hs = hs_ref[...]
    hw = hw_ref[...]

    idx = idx_ref[...]
    mask = (idx != 0).astype(jnp.float32)
    ns = jnp.sum(mask, axis=1, keepdims=True)
    ns1 = ns == 1.0
    m0 = mask[:, 0:1]
    rinv = 1.0 / jnp.where(ns1, 2.0, ns - 1.0)
    ca = jnp.where(ns1, 0.5 * m0, 0.0)
    cb = jnp.where(ns1, 0.0, rinv)
    h_r = ca * hs + cb * hw

    w_rl = wr_ref[...] - wl_ref[...]
    x = (jnp.dot(p, wt_ref[...], preferred_element_type=jnp.float32)
         + jnp.dot(hs, wl_ref[...], preferred_element_type=jnp.float32)
         + jnp.dot(h_r, w_rl, preferred_element_type=jnp.float32)
         + bias_ref[...] + p)

    mu = jnp.mean(x, axis=1, keepdims=True)
    var = jnp.mean(x * x, axis=1, keepdims=True) - mu * mu
    y = (x - mu) * jax.lax.rsqrt(var + 1e-5) * gamma_ref[...] + beta_ref[...]
    y = jnp.maximum(y, 0.0)
    out_ref[...] = jnp.max(y.reshape(gt, n, d), axis=1)


def _dense(p_flat, hs_flat, hw_flat, idx_flat, w_t, w_l, w_r, bias2, gamma2,
           beta2):
    n, d, c, gt = _N, _D, _C, _GT
    b = p_flat.shape[0] // n
    rows = gt * n
    grid = (b // gt,)
    row_spec = pl.BlockSpec((rows, d), lambda i: (i, 0))
    idx_spec = pl.BlockSpec((rows, c), lambda i: (i, 0))
    w_spec = pl.BlockSpec((d, d), lambda i: (0, 0))
    v_spec = pl.BlockSpec((1, d), lambda i: (0, 0))
    return pl.pallas_call(
        _dense_body,
        grid=grid,
        in_specs=[row_spec, row_spec, row_spec, idx_spec, w_spec, w_spec,
                  w_spec, v_spec, v_spec, v_spec],
        out_specs=pl.BlockSpec((gt, d), lambda i: (i, 0)),
        out_shape=jax.ShapeDtypeStruct((b, d), jnp.float32),
    )(p_flat, hs_flat, hw_flat, idx_flat, w_t, w_l, w_r, bias2, gamma2, beta2)


def kernel(parent_node_embedding, children_index, batch_tree_mask, w_t, w_l,
           w_r, bias, ln_gamma, ln_beta):
    del batch_tree_mask
    b, n, d, c, bsc = _B, _N, _D, _C, _B_SC
    bias2 = bias.reshape(1, d)
    gamma2 = ln_gamma.reshape(1, d)
    beta2 = ln_beta.reshape(1, d)

    p_lo = parent_node_embedding[:bsc]
    i_lo = children_index[:bsc]
    p_hi = parent_node_embedding[bsc:]
    i_hi = children_index[bsc:]

    hs2d, hw2d = _sc_gather(p_lo.reshape(bsc, n * d),
                            i_lo.reshape(bsc, n * c))

    out_hi = _tc_full(p_hi, i_hi.reshape((b - bsc) * n, c).T,
                      w_t, w_l, w_r, bias2, gamma2, beta2)

    out_lo = _dense(p_lo.reshape(bsc * n, d),
                    hs2d.reshape(bsc * n, d),
                    hw2d.reshape(bsc * n, d),
                    i_lo.reshape(bsc * n, c),
                    w_t, w_l, w_r, bias2, gamma2, beta2)

    return jnp.concatenate([out_lo, out_hi], axis=0)


# speedup vs baseline: 1.5986x; 1.1577x over previous
import functools
import jax
import jax.numpy as jnp
from jax import lax
from jax.experimental import pallas as pl
from jax.experimental.pallas import tpu as pltpu
from jax.experimental.pallas import tpu_sc as plsc

_B, _N, _C, _D = 256, 64, 32, 128
_GT = 8
_B_SC = 128
_TREES_PER_SUBCORE = _B_SC // 32
_L = 16


def _bcast_lane(vec, j):
    dn = lax.GatherDimensionNumbers(offset_dims=(), collapsed_slice_dims=(0,),
                                    start_index_map=(0,))
    idx = jnp.full((_L, 1), j, jnp.int32)
    return lax.gather(vec, idx, dn, slice_sizes=(1,),
                      mode=lax.GatherScatterMode.PROMISE_IN_BOUNDS)


def _sc_body(parent_hbm, idx_hbm, hs_hbm, hw_hbm, tree_v, idx_v, hs_v, hw_v):
    n, c, d, l = _N, _C, _D, _L
    wid = lax.axis_index("s") * 2 + lax.axis_index("c")
    zero = jnp.zeros((l,), jnp.float32)
    lane = lax.iota(jnp.int32, l)
    views = [tree_v.at[pl.ds(k * l, (n - 1) * d + l)] for k in range(d // l)]

    def tree_loop(t, carry):
        b = wid * _TREES_PER_SUBCORE + t
        pltpu.sync_copy(parent_hbm.at[b], tree_v)
        pltpu.sync_copy(idx_hbm.at[b], idx_v)
        for k in range(d // l):
            tree_v[pl.ds(k * l, l)] = zero

        def node_loop(nn, carry2):
            base = nn * c
            halves = []
            for h in range(2):
                iv = idx_v[pl.ds(base + h * l, l)]
                halves.append(iv << 7)
            accs = [jnp.zeros((l,), jnp.float32) for _ in range(d // l)]
            accw = [jnp.zeros((l,), jnp.float32) for _ in range(d // l)]
            for h, siv in reversed(list(enumerate(halves))):
                for j in reversed(range(l)):
                    cpos = h * l + j
                    addr = _bcast_lane(siv, j) + lane
                    for k in range(d // l):
                        g = plsc.load_gather(views[k], [addr])
                        accs[k] = accs[k] + g
                        if cpos != 0:
                            accw[k] = accw[k] + accs[k]
            for k in range(d // l):
                hs_v[pl.ds(nn * d + k * l, l)] = accs[k]
                hw_v[pl.ds(nn * d + k * l, l)] = accw[k]
            return carry2

        lax.fori_loop(0, n, node_loop, 0)
        pltpu.sync_copy(hs_v, hs_hbm.at[b])
        pltpu.sync_copy(hw_v, hw_hbm.at[b])
        return carry

    lax.fori_loop(0, _TREES_PER_SUBCORE, tree_loop, 0)


def _sc_gather(parent2d, idx2d):
    b, nd = parent2d.shape
    mesh = plsc.VectorSubcoreMesh(core_axis_name="c", subcore_axis_name="s")
    f = functools.partial(
        pl.kernel,
        out_type=(
            jax.ShapeDtypeStruct((b, nd), jnp.float32),
            jax.ShapeDtypeStruct((b, nd), jnp.float32),
        ),
        mesh=mesh,
        compiler_params=pltpu.CompilerParams(needs_layout_passes=False),
        scratch_types=[
            pltpu.VMEM((nd,), jnp.float32),
            pltpu.VMEM((_N * _C,), jnp.int32),
            pltpu.VMEM((nd,), jnp.float32),
            pltpu.VMEM((nd,), jnp.float32),
        ],
    )(_sc_body)
    return f(parent2d, idx2d)


def _tc_full_body(parent_ref, idxt_ref, wt_ref, wl_ref, wr_ref,
                  bias_ref, gamma_ref, beta_ref, out_ref):
    gt, n, d, c = _GT, _N, _D, _C
    rows = gt * n

    idxt = idxt_ref[...]
    mask = (idxt != 0).astype(jnp.float32)
    ns = jnp.sum(mask, axis=0, keepdims=True)
    c_iota_i = jax.lax.broadcasted_iota(jnp.int32, (c, rows), 0)
    c_iota = c_iota_i.astype(jnp.float32)
    safe = jnp.where(ns == 1.0, 1.0, ns - 1.0)
    er_gen = c_iota * mask / safe
    er_single = jnp.where(c_iota_i == 0, 0.5, 0.0)
    er = jnp.where(ns == 1.0, er_single, er_gen)
    cl = mask * (1.0 - er)
    cr = mask * er

    m_iota = jax.lax.broadcasted_iota(jnp.int32, (n, rows), 0)
    st_l = jnp.zeros((n, rows), jnp.float32)
    st_r = jnp.zeros((n, rows), jnp.float32)
    for j in range(c):
        eq = (m_iota == idxt[j:j + 1, :]).astype(jnp.float32)
        st_l = st_l + cl[j:j + 1, :] * eq
        st_r = st_r + cr[j:j + 1, :] * eq

    parent = parent_ref[...]
    dn_t = (((0,), (0,)), ((), ()))
    hl_parts = []
    hr_parts = []
    for g in range(gt):
        p_g = parent[g]
        sl_g = st_l[:, g * n:(g + 1) * n]
        sr_g = st_r[:, g * n:(g + 1) * n]
        hl_parts.append(lax.dot_general(sl_g, p_g, dn_t,
                                        preferred_element_type=jnp.float32))
        hr_parts.append(lax.dot_general(sr_g, p_g, dn_t,
                                        preferred_element_type=jnp.float32))
    h_l = jnp.concatenate(hl_parts, axis=0)
    h_r = jnp.concatenate(hr_parts, axis=0)

    p_flat = parent.reshape(rows, d)
    x = (jnp.dot(p_flat, wt_ref[...], preferred_element_type=jnp.float32)
         + jnp.dot(h_l, wl_ref[...], preferred_element_type=jnp.float32)
         + jnp.dot(h_r, wr_ref[...], preferred_element_type=jnp.float32)
         + bias_ref[...] + p_flat)

    mu = jnp.mean(x, axis=1, keepdims=True)
    var = jnp.mean(x * x, axis=1, keepdims=True) - mu * mu
    y = (x - mu) * jax.lax.rsqrt(var + 1e-5) * gamma_ref[...] + beta_ref[...]
    y = jnp.maximum(y, 0.0)
    out_ref[...] = jnp.max(y.reshape(gt, n, d), axis=1)


def _tc_full(parent3d, idx_t, w_t, w_l, w_r, bias2, gamma2, beta2):
    b, n, d, c, gt = parent3d.shape[0], _N, _D, _C, _GT
    grid = (b // gt,)
    return pl.pallas_call(
        _tc_full_body,
        grid=grid,
        in_specs=[
            pl.BlockSpec((gt, n, d), lambda i: (i, 0, 0)),
            pl.BlockSpec((c, gt * n), lambda i: (0, i)),
            pl.BlockSpec((d, d), lambda i: (0, 0)),
            pl.BlockSpec((d, d), lambda i: (0, 0)),
            pl.BlockSpec((d, d), lambda i: (0, 0)),
            pl.BlockSpec((1, d), lambda i: (0, 0)),
            pl.BlockSpec((1, d), lambda i: (0, 0)),
            pl.BlockSpec((1, d), lambda i: (0, 0)),
        ],
        out_specs=pl.BlockSpec((gt, d), lambda i: (i, 0)),
        out_shape=jax.ShapeDtypeStruct((b, d), jnp.float32),
    )(parent3d, idx_t, w_t, w_l, w_r, bias2, gamma2, beta2)


def _dense_body(p_ref, hs_ref, hw_ref, idx_ref, wt_ref, wl_ref, wr_ref,
                bias_ref, gamma_ref, beta_ref, out_ref):
    gt, n, d = _GT, _N, _D
    p = p_ref[...]
    hs = hs_ref[...]
    hw = hw_ref[...]

    idx = idx_ref[...]
    mask = (idx != 0).astype(jnp.float32)
    ns = jnp.sum(mask, axis=1, keepdims=True)
    ns1 = ns == 1.0
    m0 = mask[:, 0:1]
    rinv = 1.0 / jnp.where(ns1, 2.0, ns - 1.0)
    ca = jnp.where(ns1, 0.5 * m0, 0.0)
    cb = jnp.where(ns1, 0.0, rinv)
    h_r = ca * hs + cb * hw

    w_rl = wr_ref[...] - wl_ref[...]
    x = (jnp.dot(p, wt_ref[...], preferred_element_type=jnp.float32)
         + jnp.dot(hs, wl_ref[...], preferred_element_type=jnp.float32)
         + jnp.dot(h_r, w_rl, preferred_element_type=jnp.float32)
         + bias_ref[...] + p)

    mu = jnp.mean(x, axis=1, keepdims=True)
    var = jnp.mean(x * x, axis=1, keepdims=True) - mu * mu
    y = (x - mu) * jax.lax.rsqrt(var + 1e-5) * gamma_ref[...] + beta_ref[...]
    y = jnp.maximum(y, 0.0)
    out_ref[...] = jnp.max(y.reshape(gt, n, d), axis=1)


def _dense(p_flat, hs_flat, hw_flat, idx_flat, w_t, w_l, w_r, bias2, gamma2,
           beta2):
    n, d, c, gt = _N, _D, _C, _GT
    b = p_flat.shape[0] // n
    rows = gt * n
    grid = (b // gt,)
    row_spec = pl.BlockSpec((rows, d), lambda i: (i, 0))
    idx_spec = pl.BlockSpec((rows, c), lambda i: (i, 0))
    w_spec = pl.BlockSpec((d, d), lambda i: (0, 0))
    v_spec = pl.BlockSpec((1, d), lambda i: (0, 0))
    return pl.pallas_call(
        _dense_body,
        grid=grid,
        in_specs=[row_spec, row_spec, row_spec, idx_spec, w_spec, w_spec,
                  w_spec, v_spec, v_spec, v_spec],
        out_specs=pl.BlockSpec((gt, d), lambda i: (i, 0)),
        out_shape=jax.ShapeDtypeStruct((b, d), jnp.float32),
    )(p_flat, hs_flat, hw_flat, idx_flat, w_t, w_l, w_r, bias2, gamma2, beta2)


def kernel(parent_node_embedding, children_index, batch_tree_mask, w_t, w_l,
           w_r, bias, ln_gamma, ln_beta):
    del batch_tree_mask
    b, n, d, c, bsc = _B, _N, _D, _C, _B_SC
    bias2 = bias.reshape(1, d)
    gamma2 = ln_gamma.reshape(1, d)
    beta2 = ln_beta.reshape(1, d)

    p_lo = parent_node_embedding[:bsc]
    i_lo = children_index[:bsc]
    p_hi = parent_node_embedding[bsc:]
    i_hi = children_index[bsc:]

    hs2d, hw2d = _sc_gather(p_lo.reshape(bsc, n * d),
                            i_lo.reshape(bsc, n * c))

    out_hi = _tc_full(p_hi, i_hi.reshape((b - bsc) * n, c).T,
                      w_t, w_l, w_r, bias2, gamma2, beta2)

    out_lo = _dense(p_lo.reshape(bsc * n, d),
                    hs2d.reshape(bsc * n, d),
                    hw2d.reshape(bsc * n, d),
                    i_lo.reshape(bsc * n, c),
                    w_t, w_l, w_r, bias2, gamma2, beta2)

    return jnp.concatenate([out_lo, out_hi], axis=0)
